# baseline (device time: 20404 ns/iter reference)
import jax
import jax.numpy as jnp
from jax import lax
from jax.experimental import pallas as pl
from jax.experimental.pallas import tpu as pltpu

N_DEV = 4
BLOCK = 64
SCALE = 0.125
NEG = -1e9
SHIFT = 8.0


def kernel(x, Wq, K_ext, V_ext, Wo):
    B, S_l, D = x.shape
    Hq, Dh = K_ext.shape[2], K_ext.shape[3]
    HD = Hq * Dh
    S_h = S_l // 2

    kv = jnp.concatenate(
        [K_ext.reshape(B, S_l, HD), V_ext.reshape(B, S_l, HD)], axis=-1
    ).astype(jnp.bfloat16)
    kv4 = kv.reshape(2 * B, S_h, 2 * HD)

    def body(x_ref, wq_ref, kv4_ref, wo_ref, out_ref, kvg_ref,
             sendR_sems, sendL_sems, fwdR_sems, fwdL_sems,
             recvL_sems, recvR_sems, recvFL_sems, recvFR_sems):
        my = lax.axis_index("i")
        left = lax.rem(my + (N_DEV - 1), N_DEV)
        right = lax.rem(my + 1, N_DEV)
        far = lax.rem(my + 2, N_DEV)

        barrier_sem = pltpu.get_barrier_semaphore()
        for nbr in (left, right):
            pl.semaphore_signal(
                barrier_sem, inc=1,
                device_id=(nbr,), device_id_type=pl.DeviceIdType.MESH,
            )
        pl.semaphore_wait(barrier_sem, 2)

        def rdma(src, dst_slot, send_sem, recv_sem, dev):
            return pltpu.make_async_remote_copy(
                src_ref=src, dst_ref=kvg_ref.at[dst_slot],
                send_sem=send_sem, recv_sem=recv_sem,
                device_id=(dev,), device_id_type=pl.DeviceIdType.MESH,
            )

        sends = []
        for q in (0, 1):
            for b in range(B):
                pid = b * 2 + q
                sends.append(rdma(kv4_ref.at[pid], 4 * my + pid,
                                  sendR_sems.at[pid], recvL_sems.at[pid],
                                  right))
        for q in (1, 0):
            for b in range(B):
                pid = b * 2 + q
                sends.append(rdma(kv4_ref.at[pid], 4 * my + pid,
                                  sendL_sems.at[pid], recvR_sems.at[pid],
                                  left))
        for s in sends:
            s.start()

        row = lax.broadcasted_iota(jnp.int32, (S_l, S_l), 0)
        col = lax.broadcasted_iota(jnp.int32, (S_l, S_l), 1)
        qb = my * (S_l // BLOCK) + row // BLOCK
        kb = col // BLOCK

        wq = wq_ref[...]
        q_proj = [(jnp.dot(x_ref[b], wq, preferred_element_type=jnp.float32)
                   * SCALE).astype(jnp.bfloat16)
                  for b in range(B)]

        l = [[None] * Hq for _ in range(B)]
        acc = [[None] * Hq for _ in range(B)]

        def load_chunk(o):
            out = []
            for b in range(B):
                v = kvg_ref[pl.ds(4 * o + 2 * b, 2), :, :]
                out.append(jnp.reshape(v, (S_l, 2 * HD)))
            return out

        def process(chunk, o, first=False):
            msk = (o * (S_l // BLOCK) + kb) <= qb
            for b in range(B):
                for h in range(Hq):
                    q_h = q_proj[b][:, h * Dh:(h + 1) * Dh]
                    k_h = chunk[b][:, h * Dh:(h + 1) * Dh]
                    v_h = chunk[b][:, HD + h * Dh:HD + (h + 1) * Dh]
                    s = lax.dot_general(
                        q_h, k_h, (((1,), (1,)), ((), ())),
                        preferred_element_type=jnp.float32,
                    )
                    p = jnp.exp(jnp.where(msk, s, NEG) - SHIFT)
                    ls = jnp.sum(p, axis=-1, keepdims=True)
                    pv = jnp.dot(p.astype(jnp.bfloat16), v_h,
                                 preferred_element_type=jnp.float32)
                    if first:
                        l[b][h] = ls
                        acc[b][h] = pv
                    else:
                        l[b][h] = l[b][h] + ls
                        acc[b][h] = acc[b][h] + pv

        def process_half(o, q, first=False):
            msk = (o * (S_l // BLOCK) + q * (S_h // BLOCK)
                   + kb[:, :S_h]) <= qb[:, :S_h]
            for b in range(B):
                chunk = kvg_ref[4 * o + 2 * b + q]
                for h in range(Hq):
                    q_h = q_proj[b][:, h * Dh:(h + 1) * Dh]
                    k_h = chunk[:, h * Dh:(h + 1) * Dh]
                    v_h = chunk[:, HD + h * Dh:HD + (h + 1) * Dh]
                    s = lax.dot_general(
                        q_h, k_h, (((1,), (1,)), ((), ())),
                        preferred_element_type=jnp.float32,
                    )
                    p = jnp.exp(jnp.where(msk, s, NEG) - SHIFT)
                    ls = jnp.sum(p, axis=-1, keepdims=True)
                    pv = jnp.dot(p.astype(jnp.bfloat16), v_h,
                                 preferred_element_type=jnp.float32)
                    if first:
                        l[b][h] = ls
                        acc[b][h] = pv
                    else:
                        l[b][h] = l[b][h] + ls
                        acc[b][h] = acc[b][h] + pv

        process([jnp.reshape(kv4_ref[2 * b:2 * b + 2], (S_l, 2 * HD))
                 for b in range(B)], my, first=True)

        def recv(dst_slot, recv_sem, dev):
            return pltpu.make_async_remote_copy(
                src_ref=kv4_ref.at[0], dst_ref=kvg_ref.at[dst_slot],
                send_sem=sendR_sems.at[0], recv_sem=recv_sem,
                device_id=(dev,), device_id_type=pl.DeviceIdType.MESH,
            )

        fwds = []
        for b in range(B):
            recv(4 * left + 2 * b, recvL_sems.at[2 * b], left).wait_recv()
            f = rdma(kvg_ref.at[4 * left + 2 * b], 4 * left + 2 * b,
                     fwdR_sems.at[b], recvFL_sems.at[b], right)
            f.start()
            fwds.append(f)
        process_half(left, 0)

        for b in range(B):
            recv(4 * right + 2 * b + 1, recvR_sems.at[2 * b + 1],
                 right).wait_recv()
            f = rdma(kvg_ref.at[4 * right + 2 * b + 1],
                     4 * right + 2 * b + 1,
                     fwdL_sems.at[b], recvFR_sems.at[b], left)
            f.start()
            fwds.append(f)
        process_half(right, 1)

        for b in range(B):
            recv(4 * left + 2 * b + 1, recvL_sems.at[2 * b + 1],
                 left).wait_recv()
        process_half(left, 1)
        for b in range(B):
            recv(4 * right + 2 * b, recvR_sems.at[2 * b], right).wait_recv()
        process_half(right, 0)

        for b in range(B):
            recv(4 * far + 2 * b, recvFL_sems.at[b], left).wait_recv()
            recv(4 * far + 2 * b + 1, recvFR_sems.at[b], right).wait_recv()
        process(load_chunk(far), far)

        wo = wo_ref[...]
        for b in range(B):
            ctx = jnp.concatenate(
                [acc[b][h] / l[b][h] for h in range(Hq)], axis=-1)
            out_ref[b] = jnp.dot(ctx, wo, preferred_element_type=jnp.float32)

        for s in sends:
            s.wait_send()
        for f in fwds:
            f.wait_send()

    return pl.pallas_call(
        body,
        out_shape=jax.ShapeDtypeStruct((B, S_l, D), jnp.float32),
        in_specs=[pl.BlockSpec(memory_space=pltpu.VMEM)] * 4,
        out_specs=pl.BlockSpec(memory_space=pltpu.VMEM),
        scratch_shapes=[
            pltpu.VMEM((4 * N_DEV, S_h, 2 * HD), jnp.bfloat16),
            pltpu.SemaphoreType.DMA((4,)),
            pltpu.SemaphoreType.DMA((4,)),
            pltpu.SemaphoreType.DMA((2,)),
            pltpu.SemaphoreType.DMA((2,)),
            pltpu.SemaphoreType.DMA((4,)),
            pltpu.SemaphoreType.DMA((4,)),
            pltpu.SemaphoreType.DMA((2,)),
            pltpu.SemaphoreType.DMA((2,)),
        ],
        compiler_params=pltpu.CompilerParams(collective_id=0),
    )(x, Wq, kv4, Wo)
